# baseline (speedup 1.0000x reference)
import jax
import jax.numpy as jnp
from jax.experimental import pallas as pl
from jax.experimental.pallas import tpu as pltpu

_D = 64
_BM = 256


def _l2norm(x):
    n = jnp.sqrt(jnp.sum(x * x, axis=-1, keepdims=True))
    return x / jnp.maximum(n, 1e-12)


def _jointer_body(src_ref, tar_ref, out_ref):
    sn = _l2norm(src_ref[0])
    tn = _l2norm(tar_ref[0])
    prod = jax.lax.dot_general(
        sn, tn, (((1,), (1,)), ((), ())), preferred_element_type=jnp.float32
    )
    out_ref[0] = jnp.maximum(prod, 0.0)


def kernel(source, target, mask_src, mask_tar):
    b, n, d = source.shape
    src = source * mask_src[..., None].astype(source.dtype)
    tar = target * mask_tar[..., None].astype(target.dtype)
    out = pl.pallas_call(
        _jointer_body,
        grid=(b, n // _BM),
        in_specs=[
            pl.BlockSpec((1, _BM, d), lambda i, j: (i, j, 0)),
            pl.BlockSpec((1, n, d), lambda i, j: (i, 0, 0)),
        ],
        out_specs=pl.BlockSpec((1, _BM, n), lambda i, j: (i, j, 0)),
        out_shape=jax.ShapeDtypeStruct((b, n, n), jnp.float32),
        compiler_params=pltpu.CompilerParams(
            dimension_semantics=("parallel", "parallel"),
        ),
    )(src, tar)
    return tuple(out[i].reshape(-1) for i in range(b))


# speedup vs baseline: 1.1771x; 1.1771x over previous
import jax
import jax.numpy as jnp
from jax.experimental import pallas as pl
from jax.experimental.pallas import tpu as pltpu

_BM = 256


def _l2norm(x):
    n = jnp.sqrt(jnp.sum(x * x, axis=-1, keepdims=True))
    return x / jnp.maximum(n, 1e-12)


def _jointer_body(src_ref, tar_ref, *out_refs):
    for b, out_ref in enumerate(out_refs):
        sn = _l2norm(src_ref[b])
        tn = _l2norm(tar_ref[b])
        prod = jax.lax.dot_general(
            sn, tn, (((1,), (1,)), ((), ())), preferred_element_type=jnp.float32
        )
        out_ref[...] = jnp.maximum(prod, 0.0)


def kernel(source, target, mask_src, mask_tar):
    b, n, d = source.shape
    src = source * mask_src[..., None].astype(source.dtype)
    tar = target * mask_tar[..., None].astype(target.dtype)
    outs = pl.pallas_call(
        _jointer_body,
        grid=(n // _BM,),
        in_specs=[
            pl.BlockSpec((b, _BM, d), lambda j: (0, j, 0)),
            pl.BlockSpec((b, n, d), lambda j: (0, 0, 0)),
        ],
        out_specs=[pl.BlockSpec((_BM, n), lambda j: (j, 0)) for _ in range(b)],
        out_shape=[jax.ShapeDtypeStruct((n, n), jnp.float32) for _ in range(b)],
        compiler_params=pltpu.CompilerParams(
            dimension_semantics=("parallel",),
        ),
    )(src, tar)
    return tuple(o.reshape(-1) for o in outs)


# speedup vs baseline: 1.1909x; 1.0117x over previous
import jax
import jax.numpy as jnp
from jax.experimental import pallas as pl
from jax.experimental.pallas import tpu as pltpu

_BM = 512


def _l2norm(x):
    n = jnp.sqrt(jnp.sum(x * x, axis=-1, keepdims=True))
    return x / jnp.maximum(n, 1e-12)


def _jointer_body(src_ref, tar_ref, *out_refs):
    for b, out_ref in enumerate(out_refs):
        sn = _l2norm(src_ref[b])
        tn = _l2norm(tar_ref[b])
        prod = jax.lax.dot_general(
            sn, tn, (((1,), (1,)), ((), ())), preferred_element_type=jnp.float32
        )
        out_ref[...] = jnp.maximum(prod, 0.0)


def kernel(source, target, mask_src, mask_tar):
    b, n, d = source.shape
    src = source * mask_src[..., None].astype(source.dtype)
    tar = target * mask_tar[..., None].astype(target.dtype)
    outs = pl.pallas_call(
        _jointer_body,
        grid=(n // _BM,),
        in_specs=[
            pl.BlockSpec((b, _BM, d), lambda j: (0, j, 0)),
            pl.BlockSpec((b, n, d), lambda j: (0, 0, 0)),
        ],
        out_specs=[pl.BlockSpec((_BM, n), lambda j: (j, 0)) for _ in range(b)],
        out_shape=[jax.ShapeDtypeStruct((n, n), jnp.float32) for _ in range(b)],
        compiler_params=pltpu.CompilerParams(
            dimension_semantics=("parallel",),
        ),
    )(src, tar)
    return tuple(o.reshape(-1) for o in outs)
